# baseline (speedup 1.0000x reference)
import functools

import jax
import jax.numpy as jnp
from jax import lax
from jax.experimental import pallas as pl
from jax.experimental.pallas import tpu as pltpu
from jax.experimental.pallas import tpu_sc as plsc

B = 32
T = 2048
F = 128
D = 64
ROWS_PER_WORKER = 8
TB = 512


def _sc_gather_kernel(table0, table1, idx0, idx1, e0_out, e1_out,
                      idx_v, rows_v, sem):
    nc = 2
    wid = lax.axis_index("s") * nc + lax.axis_index("c")
    nchunks = B // ROWS_PER_WORKER

    @pl.when(wid < nchunks)
    def _():
        base = wid * ROWS_PER_WORKER
        pltpu.sync_copy(idx0.at[pl.ds(base, ROWS_PER_WORKER)], idx_v)
        pltpu.async_copy(table0.at[idx_v], rows_v, sem).wait()
        pltpu.sync_copy(rows_v, e0_out.at[pl.ds(base, ROWS_PER_WORKER)])

    @pl.when((wid >= nchunks) & (wid < 2 * nchunks))
    def _():
        base = (wid - nchunks) * ROWS_PER_WORKER
        pltpu.sync_copy(idx1.at[pl.ds(base, ROWS_PER_WORKER)], idx_v)
        pltpu.async_copy(table1.at[idx_v], rows_v, sem).wait()
        pltpu.sync_copy(rows_v, e1_out.at[pl.ds(base, ROWS_PER_WORKER)])


@functools.partial(jax.jit, static_argnums=())
def _sc_gather(table0, table1, idx0, idx1):
    mesh = plsc.VectorSubcoreMesh(core_axis_name="c", subcore_axis_name="s")
    return pl.kernel(
        _sc_gather_kernel,
        out_type=(
            jax.ShapeDtypeStruct((B, D), jnp.float32),
            jax.ShapeDtypeStruct((B, D), jnp.float32),
        ),
        mesh=mesh,
        scratch_types=[
            pltpu.VMEM((ROWS_PER_WORKER,), jnp.int32),
            pltpu.VMEM((ROWS_PER_WORKER, D), jnp.float32),
            pltpu.SemaphoreType.DMA,
        ],
        compiler_params=pltpu.CompilerParams(use_tc_tiling_on_sc=False),
    )(table0, table1, idx0, idx1)


TM = 256


def _tc_concat_kernel(spec_hbm, e0_ref, e1_ref, out_hbm, meta_buf,
                      sem_s, sem_m):
    meta = jnp.concatenate([e0_ref[...], e1_ref[...]], axis=-1)
    meta_buf[...] = jnp.broadcast_to(meta, (B, TM, 2 * D))
    nchunks = T // TM
    copies = []
    for c in range(nchunks):
        copies.append(pltpu.make_async_copy(
            spec_hbm.at[:, pl.ds(c * TM, TM), :],
            out_hbm.at[:, pl.ds(c * TM, TM), pl.ds(0, F)],
            sem_s))
        copies[-1].start()
        copies.append(pltpu.make_async_copy(
            meta_buf,
            out_hbm.at[:, pl.ds(c * TM, TM), pl.ds(F, 2 * D)],
            sem_m))
        copies[-1].start()
    for cp in copies:
        cp.wait()


def _tc_concat(spectrogram, e0, e1):
    return pl.pallas_call(
        _tc_concat_kernel,
        in_specs=[
            pl.BlockSpec(memory_space=pl.ANY),
            pl.BlockSpec(memory_space=pltpu.VMEM),
            pl.BlockSpec(memory_space=pltpu.VMEM),
        ],
        out_specs=pl.BlockSpec(memory_space=pl.ANY),
        out_shape=jax.ShapeDtypeStruct((B, T, F + 2 * D), jnp.float32),
        scratch_shapes=[
            pltpu.VMEM((B, TM, 2 * D), jnp.float32),
            pltpu.SemaphoreType.DMA,
            pltpu.SemaphoreType.DMA,
        ],
    )(spectrogram, e0.reshape(B, 1, D), e1.reshape(B, 1, D))


def kernel(spectrogram, seq_metadata, table0, table1):
    idx0 = seq_metadata[:, 0].astype(jnp.int32)
    idx1 = seq_metadata[:, 1].astype(jnp.int32)
    e0, e1 = _sc_gather(table0, table1, idx0, idx1)
    return _tc_concat(spectrogram, e0, e1)


# speedup vs baseline: 15.9235x; 15.9235x over previous
import functools

import jax
import jax.numpy as jnp
from jax import lax
from jax.experimental import pallas as pl
from jax.experimental.pallas import tpu as pltpu
from jax.experimental.pallas import tpu_sc as plsc

B = 32
T = 2048
F = 128
D = 64
ROWS_PER_WORKER = 8
TB = 256


def _sc_gather_kernel(table0, table1, idx0, idx1, e0_out, e1_out,
                      idx_v, rows_v, sem):
    nc = 2
    wid = lax.axis_index("s") * nc + lax.axis_index("c")
    nchunks = B // ROWS_PER_WORKER

    @pl.when(wid < nchunks)
    def _():
        base = wid * ROWS_PER_WORKER
        pltpu.sync_copy(idx0.at[pl.ds(base, ROWS_PER_WORKER)], idx_v)
        pltpu.async_copy(table0.at[idx_v], rows_v, sem).wait()
        pltpu.sync_copy(rows_v, e0_out.at[pl.ds(base, ROWS_PER_WORKER)])

    @pl.when((wid >= nchunks) & (wid < 2 * nchunks))
    def _():
        base = (wid - nchunks) * ROWS_PER_WORKER
        pltpu.sync_copy(idx1.at[pl.ds(base, ROWS_PER_WORKER)], idx_v)
        pltpu.async_copy(table1.at[idx_v], rows_v, sem).wait()
        pltpu.sync_copy(rows_v, e1_out.at[pl.ds(base, ROWS_PER_WORKER)])


@functools.partial(jax.jit, static_argnums=())
def _sc_gather(table0, table1, idx0, idx1):
    mesh = plsc.VectorSubcoreMesh(core_axis_name="c", subcore_axis_name="s")
    return pl.kernel(
        _sc_gather_kernel,
        out_type=(
            jax.ShapeDtypeStruct((B, D), jnp.float32),
            jax.ShapeDtypeStruct((B, D), jnp.float32),
        ),
        mesh=mesh,
        scratch_types=[
            pltpu.VMEM((ROWS_PER_WORKER,), jnp.int32),
            pltpu.VMEM((ROWS_PER_WORKER, D), jnp.float32),
            pltpu.SemaphoreType.DMA,
        ],
        compiler_params=pltpu.CompilerParams(use_tc_tiling_on_sc=False),
    )(table0, table1, idx0, idx1)


NSTEP = T // TB


def _in_cp(spec_ref, in_buf, sem_in, s, slot):
    return pltpu.make_async_copy(
        spec_ref.at[:, pl.ds(s * TB, TB), :], in_buf.at[slot], sem_in.at[slot])


def _out_cp(out_buf, out_ref, sem_out, s, slot):
    return pltpu.make_async_copy(
        out_buf.at[slot], out_ref.at[:, pl.ds(s * TB, TB), :],
        sem_out.at[slot])


def _tc_concat_kernel(spec_ref, e0_ref, e1_ref, out_ref,
                      in_buf, out_buf, sem_in, sem_out):
    s = pl.program_id(0)
    slot = lax.rem(s, 2)

    @pl.when(s == 0)
    def _():
        _in_cp(spec_ref, in_buf, sem_in, 0, 0).start()

    @pl.when(s + 1 < NSTEP)
    def _():
        _in_cp(spec_ref, in_buf, sem_in, s + 1, lax.rem(s + 1, 2)).start()

    @pl.when(s >= 2)
    def _():
        _out_cp(out_buf, out_ref, sem_out, s - 2, slot).wait()

    _in_cp(spec_ref, in_buf, sem_in, s, slot).wait()

    out_buf[slot, :, :, 0:F] = in_buf[slot]
    meta = jnp.concatenate([e0_ref[...], e1_ref[...]], axis=-1)
    out_buf[slot, :, :, F:F + 2 * D] = jnp.broadcast_to(meta, (B, TB, 2 * D))

    _out_cp(out_buf, out_ref, sem_out, s, slot).start()

    @pl.when(s == NSTEP - 1)
    def _():
        @pl.when(s >= 1)
        def _():
            _out_cp(out_buf, out_ref, sem_out, s - 1, lax.rem(s + 1, 2)).wait()
        _out_cp(out_buf, out_ref, sem_out, s, slot).wait()


def _tc_concat(spectrogram, e0, e1):
    return pl.pallas_call(
        _tc_concat_kernel,
        grid=(NSTEP,),
        in_specs=[
            pl.BlockSpec(memory_space=pl.ANY),
            pl.BlockSpec(memory_space=pltpu.VMEM),
            pl.BlockSpec(memory_space=pltpu.VMEM),
        ],
        out_specs=pl.BlockSpec(memory_space=pl.ANY),
        out_shape=jax.ShapeDtypeStruct((B, T, F + 2 * D), jnp.float32),
        scratch_shapes=[
            pltpu.VMEM((2, B, TB, F), jnp.float32),
            pltpu.VMEM((2, B, TB, F + 2 * D), jnp.float32),
            pltpu.SemaphoreType.DMA((2,)),
            pltpu.SemaphoreType.DMA((2,)),
        ],
        compiler_params=pltpu.CompilerParams(
            dimension_semantics=("arbitrary",),
        ),
    )(spectrogram, e0.reshape(B, 1, D), e1.reshape(B, 1, D))


def kernel(spectrogram, seq_metadata, table0, table1):
    idx0 = seq_metadata[:, 0].astype(jnp.int32)
    idx1 = seq_metadata[:, 1].astype(jnp.int32)
    e0, e1 = _sc_gather(table0, table1, idx0, idx1)
    return _tc_concat(spectrogram, e0, e1)
